# baseline (speedup 1.0000x reference)
import functools

import jax
import jax.numpy as jnp
from jax.experimental import pallas as pl

_NC = 2


def _relayout_body(f_ref, out_ref, *, H, W, C):
    HW = H * W
    CB = C // _NC
    feat_t = jnp.transpose(f_ref[0].reshape(CB, HW)).reshape(H, W, CB)
    out_ref[0] = jnp.zeros((H + 2, W + 2, CB), jnp.float32)
    out_ref[0, 1:H + 1, 1:W + 1, :] = feat_t


def _conv_topk_body(fp_ref, w1_ref, b1_ref, w2_ref, b2_ref, out_ref, *, H, W, C, nq):
    HW = H * W
    fp = fp_ref[0]
    acc = jnp.zeros((HW, C), jnp.float32)
    for t in range(9):
        kh, kw = t // 3, t % 3
        a = fp[kh:kh + H, kw:kw + W, :].reshape(HW, C)
        acc = acc + jnp.dot(a, w1_ref[t], preferred_element_type=jnp.float32)
    x1 = jnp.maximum(acc + b1_ref[0][None, :], 0.0)
    logits = jnp.dot(x1, w2_ref[...], preferred_element_type=jnp.float32)
    pred = jax.nn.sigmoid(logits + b2_ref[0][None, :])

    s_col = pred[:, 4:5]
    s_row = jnp.transpose(pred)[4:5, :]

    rank = jnp.zeros((1, HW), jnp.int32)
    CHUNK = 256
    i_iota = jax.lax.broadcasted_iota(jnp.int32, (CHUNK, HW), 1)
    j_iota_base = jax.lax.broadcasted_iota(jnp.int32, (CHUNK, HW), 0)
    s_i = jnp.broadcast_to(s_row, (CHUNK, HW))
    for j0 in range(0, HW, CHUNK):
        s_j = jnp.broadcast_to(s_col[j0:j0 + CHUNK], (CHUNK, HW))
        j_iota = j_iota_base + j0
        m = (s_j > s_i) | ((s_j == s_i) & (j_iota < i_iota))
        rank = rank + jnp.sum(m.astype(jnp.int32), axis=0, keepdims=True)

    r_iota = jax.lax.broadcasted_iota(jnp.int32, (nq, HW), 0)
    onehot = (jnp.broadcast_to(rank, (nq, HW)) == r_iota).astype(jnp.float32)
    out_ref[0] = jnp.dot(onehot, pred, preferred_element_type=jnp.float32)


def kernel(features, W1, b1, W2, b2, det_emb, rec_emb):
    B, C, H, W = features.shape
    nq = det_emb.shape[0]
    CB = C // _NC

    w1 = jnp.transpose(W1, (2, 3, 1, 0)).reshape(9, C, C)
    w2 = jnp.pad(jnp.transpose(W2[:, :, 0, 0]), ((0, 0), (0, 123)))
    b1r = b1.reshape(1, C)
    b2r = jnp.pad(b2, (0, 123)).reshape(1, 128)

    f_pad = pl.pallas_call(
        functools.partial(_relayout_body, H=H, W=W, C=C),
        grid=(B, _NC),
        in_specs=[pl.BlockSpec((1, CB, H, W), lambda b, c: (b, c, 0, 0))],
        out_specs=pl.BlockSpec((1, H + 2, W + 2, CB), lambda b, c: (b, 0, 0, c)),
        out_shape=jax.ShapeDtypeStruct((B, H + 2, W + 2, C), jnp.float32),
    )(features)

    out = pl.pallas_call(
        functools.partial(_conv_topk_body, H=H, W=W, C=C, nq=nq),
        grid=(B,),
        in_specs=[
            pl.BlockSpec((1, H + 2, W + 2, C), lambda b: (b, 0, 0, 0)),
            pl.BlockSpec((9, C, C), lambda b: (0, 0, 0)),
            pl.BlockSpec((1, C), lambda b: (0, 0)),
            pl.BlockSpec((C, 128), lambda b: (0, 0)),
            pl.BlockSpec((1, 128), lambda b: (0, 0)),
        ],
        out_specs=pl.BlockSpec((1, nq, 128), lambda b: (b, 0, 0)),
        out_shape=jax.ShapeDtypeStruct((B, nq, 128), jnp.float32),
    )(f_pad, w1, b1r, w2, b2r)

    coarse = out[:, :, :5]
    det_queries = jnp.broadcast_to(det_emb[None, :, :], (B, nq, C))
    rec_queries = jnp.broadcast_to(rec_emb[None, :, :], (B, nq, C))
    return (det_queries, rec_queries, coarse)


# speedup vs baseline: 1.4350x; 1.4350x over previous
import functools

import jax
import jax.numpy as jnp
from jax.experimental import pallas as pl


def _body(fp_ref, w1_ref, b1_ref, w2_ref, b2_ref, out_ref, *, H, W, C, nq):
    HW = H * W
    fp = fp_ref[0]
    fpw = [fp[:, kw:kw + W, :] for kw in range(3)]
    acc = jnp.zeros((HW, C), jnp.float32)
    for t in range(9):
        kh, kw = t // 3, t % 3
        a = fpw[kw][kh:kh + H].reshape(HW, C)
        acc = acc + jnp.dot(a, w1_ref[t], preferred_element_type=jnp.float32)
    x1 = jnp.maximum(acc + b1_ref[0][None, :], 0.0)
    logits = jnp.dot(x1, w2_ref[...], preferred_element_type=jnp.float32)
    pred = jax.nn.sigmoid(logits + b2_ref[0][None, :])

    s_col = pred[:, 4:5]
    s_row = jnp.transpose(pred)[4:5, :]

    rank = jnp.zeros((1, HW), jnp.int32)
    CHUNK = 256
    i_iota = jax.lax.broadcasted_iota(jnp.int32, (CHUNK, HW), 1)
    j_iota_base = jax.lax.broadcasted_iota(jnp.int32, (CHUNK, HW), 0)
    s_i = jnp.broadcast_to(s_row, (CHUNK, HW))
    for j0 in range(0, HW, CHUNK):
        s_j = jnp.broadcast_to(s_col[j0:j0 + CHUNK], (CHUNK, HW))
        j_iota = j_iota_base + j0
        m = (s_j > s_i) | ((s_j == s_i) & (j_iota < i_iota))
        rank = rank + jnp.sum(m.astype(jnp.int32), axis=0, keepdims=True)

    r_iota = jax.lax.broadcasted_iota(jnp.int32, (nq, HW), 0)
    onehot = (jnp.broadcast_to(rank, (nq, HW)) == r_iota).astype(jnp.float32)
    out_ref[0] = jnp.dot(onehot, pred, preferred_element_type=jnp.float32)


def kernel(features, W1, b1, W2, b2, det_emb, rec_emb):
    B, C, H, W = features.shape
    nq = det_emb.shape[0]

    f_nhwc = jnp.transpose(features, (0, 2, 3, 1))
    f_pad = jnp.pad(f_nhwc, ((0, 0), (1, 1), (1, 1), (0, 0)))
    w1 = jnp.transpose(W1, (2, 3, 1, 0)).reshape(9, C, C)
    w2 = jnp.pad(jnp.transpose(W2[:, :, 0, 0]), ((0, 0), (0, 123)))
    b1r = b1.reshape(1, C)
    b2r = jnp.pad(b2, (0, 123)).reshape(1, 128)

    out = pl.pallas_call(
        functools.partial(_body, H=H, W=W, C=C, nq=nq),
        grid=(B,),
        in_specs=[
            pl.BlockSpec((1, H + 2, W + 2, C), lambda b: (b, 0, 0, 0)),
            pl.BlockSpec((9, C, C), lambda b: (0, 0, 0)),
            pl.BlockSpec((1, C), lambda b: (0, 0)),
            pl.BlockSpec((C, 128), lambda b: (0, 0)),
            pl.BlockSpec((1, 128), lambda b: (0, 0)),
        ],
        out_specs=pl.BlockSpec((1, nq, 128), lambda b: (b, 0, 0)),
        out_shape=jax.ShapeDtypeStruct((B, nq, 128), jnp.float32),
    )(f_pad, w1, b1r, w2, b2r)

    coarse = out[:, :, :5]
    det_queries = jnp.broadcast_to(det_emb[None, :, :], (B, nq, C))
    rec_queries = jnp.broadcast_to(rec_emb[None, :, :], (B, nq, C))
    return (det_queries, rec_queries, coarse)


# speedup vs baseline: 1.4378x; 1.0019x over previous
import functools

import jax
import jax.numpy as jnp
from jax.experimental import pallas as pl


def _body(fp_ref, w1_ref, b1_ref, w2_ref, b2_ref, out_ref, *, H, W, C, nq):
    HW = H * W
    fp = fp_ref[0]
    fpw = [fp[:, kw:kw + W, :] for kw in range(3)]
    acc = jnp.zeros((HW, C), jnp.float32)
    for t in range(9):
        kh, kw = t // 3, t % 3
        a = fpw[kw][kh:kh + H].reshape(HW, C)
        acc = acc + jnp.dot(a, w1_ref[t], preferred_element_type=jnp.float32)
    x1 = jnp.maximum(acc + b1_ref[0][None, :], 0.0)
    logits = jnp.dot(x1, w2_ref[...], preferred_element_type=jnp.float32)
    logits = logits + b2_ref[0][None, :]

    s_col = logits[:, 4:5]
    s_row = jnp.transpose(logits)[4:5, :]

    rank = jnp.zeros((1, HW), jnp.int32)
    CHUNK = 256
    i_iota = jax.lax.broadcasted_iota(jnp.int32, (CHUNK, HW), 1)
    j_iota_base = jax.lax.broadcasted_iota(jnp.int32, (CHUNK, HW), 0)
    s_i = jnp.broadcast_to(s_row, (CHUNK, HW))
    for j0 in range(0, HW, CHUNK):
        s_j = jnp.broadcast_to(s_col[j0:j0 + CHUNK], (CHUNK, HW))
        j_iota = j_iota_base + j0
        m = (s_j > s_i) | ((s_j == s_i) & (j_iota < i_iota))
        rank = rank + jnp.sum(m.astype(jnp.int32), axis=0, keepdims=True)

    r_iota = jax.lax.broadcasted_iota(jnp.int32, (nq, HW), 0)
    onehot = (jnp.broadcast_to(rank, (nq, HW)) == r_iota).astype(jnp.float32)
    top = jnp.dot(onehot, logits, preferred_element_type=jnp.float32)
    out_ref[0] = jax.nn.sigmoid(top)


def kernel(features, W1, b1, W2, b2, det_emb, rec_emb):
    B, C, H, W = features.shape
    nq = det_emb.shape[0]

    f_nhwc = jnp.transpose(features, (0, 2, 3, 1))
    f_pad = jnp.pad(f_nhwc, ((0, 0), (1, 1), (1, 1), (0, 0)))
    w1 = jnp.transpose(W1, (2, 3, 1, 0)).reshape(9, C, C)
    w2 = jnp.pad(jnp.transpose(W2[:, :, 0, 0]), ((0, 0), (0, 123)))
    b1r = b1.reshape(1, C)
    b2r = jnp.pad(b2, (0, 123)).reshape(1, 128)

    out = pl.pallas_call(
        functools.partial(_body, H=H, W=W, C=C, nq=nq),
        grid=(B,),
        in_specs=[
            pl.BlockSpec((1, H + 2, W + 2, C), lambda b: (b, 0, 0, 0)),
            pl.BlockSpec((9, C, C), lambda b: (0, 0, 0)),
            pl.BlockSpec((1, C), lambda b: (0, 0)),
            pl.BlockSpec((C, 128), lambda b: (0, 0)),
            pl.BlockSpec((1, 128), lambda b: (0, 0)),
        ],
        out_specs=pl.BlockSpec((1, nq, 128), lambda b: (b, 0, 0)),
        out_shape=jax.ShapeDtypeStruct((B, nq, 128), jnp.float32),
    )(f_pad, w1, b1r, w2, b2r)

    coarse = out[:, :, :5]
    det_queries = jnp.broadcast_to(det_emb[None, :, :], (B, nq, C))
    rec_queries = jnp.broadcast_to(rec_emb[None, :, :], (B, nq, C))
    return (det_queries, rec_queries, coarse)
